# baseline (speedup 1.0000x reference)
import functools

import jax
import jax.numpy as jnp
from jax import lax
from jax.experimental import pallas as pl
from jax.experimental.pallas import tpu as pltpu
from jax.experimental.pallas import tpu_sc as plsc

BATCH = 16384
D = 64
NUM_CORES = 2
NUM_SUBCORES = 16
NUM_WORKERS = NUM_CORES * NUM_SUBCORES
BPW = BATCH // NUM_WORKERS
CHUNK = 128
NCHUNK = BPW // CHUNK


def _dot_body(uidx_hbm, iidx_hbm, utab_hbm, itab_hbm, out_hbm,
              uix_v, iix_v, urows_v, irows_v, out_v, sem):
    wid = lax.axis_index("s") * NUM_CORES + lax.axis_index("c")
    base = wid * BPW

    pltpu.sync_copy(uidx_hbm.at[pl.ds(base, BPW)], uix_v)
    pltpu.sync_copy(iidx_hbm.at[pl.ds(base, BPW)], iix_v)

    copies = []
    for c in range(NCHUNK):
        sl = pl.ds(c * CHUNK, CHUNK)
        copies.append(pltpu.async_copy(utab_hbm.at[uix_v.at[sl]],
                                       urows_v.at[sl], sem))
        copies.append(pltpu.async_copy(itab_hbm.at[iix_v.at[sl]],
                                       irows_v.at[sl], sem))

    lane_iota = lax.iota(jnp.int32, 16)
    groups_per_chunk = CHUNK // 16

    for c in range(NCHUNK):
        copies[2 * c].wait()
        copies[2 * c + 1].wait()

        def group(g, carry):
            r0 = g * 16

            def row(k, resvec):
                r = r0 + k
                a0 = urows_v[r, pl.ds(0, 16)] * irows_v[r, pl.ds(0, 16)]
                a1 = urows_v[r, pl.ds(16, 16)] * irows_v[r, pl.ds(16, 16)]
                a2 = urows_v[r, pl.ds(32, 16)] * irows_v[r, pl.ds(32, 16)]
                a3 = urows_v[r, pl.ds(48, 16)] * irows_v[r, pl.ds(48, 16)]
                acc = (a0 + a1) + (a2 + a3)
                for sh in (8, 4, 2, 1):
                    shuf = lax.gather(
                        acc, (lane_iota ^ sh)[:, None],
                        dimension_numbers=lax.GatherDimensionNumbers(
                            offset_dims=(), collapsed_slice_dims=(0,),
                            start_index_map=(0,)),
                        slice_sizes=(1,),
                        mode=lax.GatherScatterMode.PROMISE_IN_BOUNDS)
                    acc = acc + shuf
                return jnp.where(lane_iota == k, acc, resvec)

            resvec = lax.fori_loop(0, 16, row, jnp.zeros((16,), jnp.float32),
                                   unroll=16)
            out_v[pl.ds(r0, 16)] = resvec
            return carry

        lax.fori_loop(c * groups_per_chunk, (c + 1) * groups_per_chunk,
                      group, 0)

    pltpu.sync_copy(out_v, out_hbm.at[pl.ds(base, BPW)])


@jax.jit
def _mf_predict(u_idx, i_idx, users_weight, items_weight):
    mesh = plsc.VectorSubcoreMesh(core_axis_name="c", subcore_axis_name="s")
    f = functools.partial(
        pl.kernel,
        mesh=mesh,
        out_type=jax.ShapeDtypeStruct((BATCH,), jnp.float32),
        scratch_types=[
            pltpu.VMEM((BPW,), jnp.int32),
            pltpu.VMEM((BPW,), jnp.int32),
            pltpu.VMEM((BPW, D), jnp.float32),
            pltpu.VMEM((BPW, D), jnp.float32),
            pltpu.VMEM((BPW,), jnp.float32),
            pltpu.SemaphoreType.DMA,
        ],
        compiler_params=pltpu.CompilerParams(use_tc_tiling_on_sc=False),
    )(_dot_body)
    return f(u_idx, i_idx, users_weight, items_weight)


def kernel(x, users_weight, items_weight):
    u_idx = x[:, 0].astype(jnp.int32)
    i_idx = x[:, 1].astype(jnp.int32)
    return _mf_predict(u_idx, i_idx, users_weight, items_weight)


# speedup vs baseline: 1.5612x; 1.5612x over previous
import functools

import jax
import jax.numpy as jnp
from jax import lax
from jax.experimental import pallas as pl
from jax.experimental.pallas import tpu as pltpu
from jax.experimental.pallas import tpu_sc as plsc

BATCH = 16384
D = 64
NUM_CORES = 2
NUM_SUBCORES = 16
NUM_WORKERS = NUM_CORES * NUM_SUBCORES
BPW = BATCH // NUM_WORKERS
BLK = 64
NBLK = BPW // BLK


def _dot_body(uidx_hbm, iidx_hbm, utab_hbm, itab_hbm, out_hbm,
              uix_v, iix_v, slab_u, slab_i, out_v, sem_a, sem_b):
    wid = lax.axis_index("s") * NUM_CORES + lax.axis_index("c")
    base = wid * BPW

    pltpu.sync_copy(uidx_hbm.at[pl.ds(base, BPW)], uix_v)
    pltpu.sync_copy(iidx_hbm.at[pl.ds(base, BPW)], iix_v)

    sems = (sem_a, sem_b)
    lane_iota = lax.iota(jnp.int32, 16)

    def issue(b):
        buf = b & 1
        sem = sems[buf]

        def grp(g, carry):
            gbase = b * BLK + g * 16
            uvec = uix_v[pl.ds(gbase, 16)]
            ivec = iix_v[pl.ds(gbase, 16)]
            for k in range(16):
                r = g * 16 + k
                pltpu.async_copy(utab_hbm.at[uvec[k]], slab_u.at[buf, r], sem)
                pltpu.async_copy(itab_hbm.at[ivec[k]], slab_i.at[buf, r], sem)
            return carry

        lax.fori_loop(0, BLK // 16, grp, 0)

    def drain(b):
        buf = b & 1
        sem = sems[buf]
        pltpu.make_async_copy(utab_hbm.at[pl.ds(0, BLK)],
                              slab_u.at[buf], sem).wait()
        pltpu.make_async_copy(itab_hbm.at[pl.ds(0, BLK)],
                              slab_i.at[buf], sem).wait()

    def compute(b):
        buf = b & 1

        def group(g, carry):
            def row(k, resvec):
                r = g * 16 + k
                a0 = slab_u[buf, r, pl.ds(0, 16)] * slab_i[buf, r, pl.ds(0, 16)]
                a1 = slab_u[buf, r, pl.ds(16, 16)] * slab_i[buf, r, pl.ds(16, 16)]
                a2 = slab_u[buf, r, pl.ds(32, 16)] * slab_i[buf, r, pl.ds(32, 16)]
                a3 = slab_u[buf, r, pl.ds(48, 16)] * slab_i[buf, r, pl.ds(48, 16)]
                acc = (a0 + a1) + (a2 + a3)
                for sh in (8, 4, 2, 1):
                    shuf = lax.gather(
                        acc, (lane_iota ^ sh)[:, None],
                        dimension_numbers=lax.GatherDimensionNumbers(
                            offset_dims=(), collapsed_slice_dims=(0,),
                            start_index_map=(0,)),
                        slice_sizes=(1,),
                        mode=lax.GatherScatterMode.PROMISE_IN_BOUNDS)
                    acc = acc + shuf
                return jnp.where(lane_iota == k, acc, resvec)

            resvec = lax.fori_loop(0, 16, row, jnp.zeros((16,), jnp.float32),
                                   unroll=16)
            out_v[pl.ds(b * BLK + g * 16, 16)] = resvec
            return carry

        lax.fori_loop(0, BLK // 16, group, 0)

    issue(0)
    for b in range(NBLK):
        if b + 1 < NBLK:
            issue(b + 1)
        drain(b)
        compute(b)

    pltpu.sync_copy(out_v, out_hbm.at[pl.ds(base, BPW)])


@jax.jit
def _mf_predict(u_idx, i_idx, users_weight, items_weight):
    mesh = plsc.VectorSubcoreMesh(core_axis_name="c", subcore_axis_name="s")
    f = functools.partial(
        pl.kernel,
        mesh=mesh,
        out_type=jax.ShapeDtypeStruct((BATCH,), jnp.float32),
        scratch_types=[
            pltpu.VMEM((BPW,), jnp.int32),
            pltpu.VMEM((BPW,), jnp.int32),
            pltpu.VMEM((2, BLK, D), jnp.float32),
            pltpu.VMEM((2, BLK, D), jnp.float32),
            pltpu.VMEM((BPW,), jnp.float32),
            pltpu.SemaphoreType.DMA,
            pltpu.SemaphoreType.DMA,
        ],
    )(_dot_body)
    return f(u_idx, i_idx, users_weight, items_weight)


def kernel(x, users_weight, items_weight):
    u_idx = x[:, 0].astype(jnp.int32)
    i_idx = x[:, 1].astype(jnp.int32)
    return _mf_predict(u_idx, i_idx, users_weight, items_weight)
